# baseline (speedup 1.0000x reference)
import jax
import jax.numpy as jnp
from jax.experimental import pallas as pl

T = 2048
D = 1024
NH = 16
NKV = 4
HD = 64
E = 8
TOPK = 2
NG = 4
TG = 2
DFF = 512
EPS = 1e-05

BT = 256
NBT = T // BT

_HIGH = jax.lax.Precision.HIGHEST


def _rms(x, w, eps=EPS):
    v = jnp.mean(x * x, axis=-1, keepdims=True)
    return x * jax.lax.rsqrt(v + eps) * w


def _dot_nt(a, b):
    return jax.lax.dot_general(
        a.astype(jnp.bfloat16), b.astype(jnp.bfloat16),
        (((1,), (1,)), ((), ())), preferred_element_type=jnp.float32)


def _k1_body(h_ref, ln1_ref, qkvw_ref, gatew_ref, qkv_out, gate_out):
    x = h_ref[...]
    xn = _rms(x, ln1_ref[...])
    qkv_out[...] = _dot_nt(xn, qkvw_ref[...])
    gate_out[...] = _dot_nt(xn, gatew_ref[...])


def _k1(h, ln1_w, qkv_w, attn_gate_w):
    return pl.pallas_call(
        _k1_body,
        grid=(NBT,),
        in_specs=[
            pl.BlockSpec((BT, D), lambda i: (i, 0)),
            pl.BlockSpec((1, D), lambda i: (0, 0)),
            pl.BlockSpec((NH * HD + 2 * NKV * HD, D), lambda i: (0, 0)),
            pl.BlockSpec((NH * HD, D), lambda i: (0, 0)),
        ],
        out_specs=[
            pl.BlockSpec((BT, NH * HD + 2 * NKV * HD), lambda i: (i, 0)),
            pl.BlockSpec((BT, NH * HD), lambda i: (i, 0)),
        ],
        out_shape=[
            jax.ShapeDtypeStruct((T, NH * HD + 2 * NKV * HD), jnp.float32),
            jax.ShapeDtypeStruct((T, NH * HD), jnp.float32),
        ],
    )(h, ln1_w.reshape(1, D), qkv_w, attn_gate_w)


def _attn_body(q_ref, k_ref, v_ref, qn_ref, kn_ref, o_ref):
    i = pl.program_id(1)
    qb = _rms(q_ref[0], qn_ref[...]) * (HD ** -0.5)
    qb = qb.astype(jnp.bfloat16)
    rows = jax.lax.broadcasted_iota(jnp.int32, (BT, BT), 0)
    cols = jax.lax.broadcasted_iota(jnp.int32, (BT, BT), 1)

    def body(j, carry):
        m, l, acc = carry
        kb = _rms(k_ref[0, pl.ds(j * BT, BT), :], kn_ref[...])
        s = jax.lax.dot_general(
            qb, kb.astype(jnp.bfloat16), (((1,), (1,)), ((), ())),
            preferred_element_type=jnp.float32)
        mask = (j * BT + cols) <= (i * BT + rows)
        s = jnp.where(mask, s, -1e30)
        m_new = jnp.maximum(m, jnp.max(s, axis=-1, keepdims=True))
        alpha = jnp.exp(m - m_new)
        p = jnp.exp(s - m_new)
        l = l * alpha + jnp.sum(p, axis=-1, keepdims=True)
        vb = v_ref[0, pl.ds(j * BT, BT), :]
        acc = acc * alpha + jax.lax.dot_general(
            p.astype(jnp.bfloat16), vb.astype(jnp.bfloat16),
            (((1,), (0,)), ((), ())), preferred_element_type=jnp.float32)
        return m_new, l, acc

    m0 = jnp.full((BT, 1), -1e30, jnp.float32)
    l0 = jnp.zeros((BT, 1), jnp.float32)
    a0 = jnp.zeros((BT, HD), jnp.float32)
    m, l, acc = jax.lax.fori_loop(0, i + 1, body, (m0, l0, a0))
    o_ref[0] = acc / l


def _k2(q, k, v, q_norm_w, k_norm_w):
    rep = NH // NKV
    return pl.pallas_call(
        _attn_body,
        grid=(NH, NBT),
        in_specs=[
            pl.BlockSpec((1, BT, HD), lambda h, i: (h, i, 0)),
            pl.BlockSpec((1, T, HD), lambda h, i: (h // rep, 0, 0)),
            pl.BlockSpec((1, T, HD), lambda h, i: (h // rep, 0, 0)),
            pl.BlockSpec((1, HD), lambda h, i: (0, 0)),
            pl.BlockSpec((1, HD), lambda h, i: (0, 0)),
        ],
        out_specs=pl.BlockSpec((1, BT, HD), lambda h, i: (h, i, 0)),
        out_shape=jax.ShapeDtypeStruct((NH, T, HD), jnp.float32),
    )(q, k, v, q_norm_w.reshape(1, HD), k_norm_w.reshape(1, HD))


def _perm_mat(n, r):
    a = jax.lax.broadcasted_iota(jnp.int32, (n, n), 0)
    e = jax.lax.broadcasted_iota(jnp.int32, (n, n), 1)
    tgt = e + r - n * ((e + r) >= n).astype(jnp.int32)
    return (a == tgt).astype(jnp.float32)


def _rank_lt(m, kmax, n):
    rank = jnp.zeros_like(m)
    e_idx = jax.lax.broadcasted_iota(jnp.int32, m.shape, 1)
    for r in range(1, n):
        m_rot = jax.lax.dot_general(
            m, _perm_mat(n, r), (((1,), (0,)), ((), ())),
            preferred_element_type=jnp.float32, precision=_HIGH)
        beat = (m_rot > m) | ((m_rot == m) & (e_idx >= n - r))
        rank = rank + beat.astype(jnp.float32)
    return (rank < kmax).astype(jnp.float32)


def _k3_body(o_ref, gate_ref, res_ref, ow_ref, ln2_ref, rw_ref, bias_ref,
             h2_out, hn2_out, cmb_out):
    og = o_ref[...] * jax.nn.sigmoid(gate_ref[...])
    h2 = _dot_nt(og, ow_ref[...]) + res_ref[...]
    h2_out[...] = h2
    hn2 = _rms(h2, ln2_ref[...])
    hn2_out[...] = hn2.astype(jnp.bfloat16)

    logits = jax.lax.dot_general(
        hn2, rw_ref[...], (((1,), (1,)), ((), ())),
        preferred_element_type=jnp.float32, precision=_HIGH)
    scores = jax.nn.sigmoid(logits)
    sfc = scores + bias_ref[...]

    epg = E // NG
    a = jax.lax.broadcasted_iota(jnp.int32, (E, NG), 0)
    g = jax.lax.broadcasted_iota(jnp.int32, (E, NG), 1)
    pair = ((a // epg) == g).astype(jnp.float32)
    gs = jax.lax.dot_general(
        sfc, pair, (((1,), (0,)), ((), ())),
        preferred_element_type=jnp.float32, precision=_HIGH)

    sel_g = _rank_lt(gs, TG, NG)
    ge = jax.lax.broadcasted_iota(jnp.int32, (NG, E), 0)
    ee = jax.lax.broadcasted_iota(jnp.int32, (NG, E), 1)
    expand = (ge == (ee // epg)).astype(jnp.float32)
    mask_e = jax.lax.dot_general(
        sel_g, expand, (((1,), (0,)), ((), ())),
        preferred_element_type=jnp.float32, precision=_HIGH)

    masked = jnp.where(mask_e > 0.5, sfc, -1e30)
    sel_e = _rank_lt(masked, TOPK, E)
    w = scores * sel_e
    denom = jnp.sum(w, axis=-1, keepdims=True) + 1e-20
    cmb_out[...] = w / denom


def _k3(o2, gate, res, o_w, ln2_w, router_w, expert_bias):
    return pl.pallas_call(
        _k3_body,
        grid=(NBT,),
        in_specs=[
            pl.BlockSpec((BT, D), lambda i: (i, 0)),
            pl.BlockSpec((BT, D), lambda i: (i, 0)),
            pl.BlockSpec((BT, D), lambda i: (i, 0)),
            pl.BlockSpec((D, D), lambda i: (0, 0)),
            pl.BlockSpec((1, D), lambda i: (0, 0)),
            pl.BlockSpec((E, D), lambda i: (0, 0)),
            pl.BlockSpec((1, E), lambda i: (0, 0)),
        ],
        out_specs=[
            pl.BlockSpec((BT, D), lambda i: (i, 0)),
            pl.BlockSpec((BT, D), lambda i: (i, 0)),
            pl.BlockSpec((BT, E), lambda i: (i, 0)),
        ],
        out_shape=[
            jax.ShapeDtypeStruct((T, D), jnp.float32),
            jax.ShapeDtypeStruct((T, D), jnp.bfloat16),
            jax.ShapeDtypeStruct((T, E), jnp.float32),
        ],
    )(o2, gate, res, o_w, ln2_w.reshape(1, D), router_w,
      expert_bias.reshape(1, E))


def _k4_body(x_ref, res_ref, cmb_ref, wg_ref, wu_ref, wd_ref,
             sg_ref, su_ref, sd_ref, out_ref):
    x = x_ref[...]
    cmb = cmb_ref[...]
    col = jax.lax.broadcasted_iota(jnp.int32, (BT, E), 1)
    out = res_ref[...]

    def ffn(wg, wu, wd):
        g = jax.lax.dot_general(x, wg, (((1,), (1,)), ((), ())),
                                preferred_element_type=jnp.float32)
        u = jax.lax.dot_general(x, wu, (((1,), (1,)), ((), ())),
                                preferred_element_type=jnp.float32)
        a = (g * jax.nn.sigmoid(g) * u).astype(jnp.bfloat16)
        return jax.lax.dot_general(a, wd, (((1,), (1,)), ((), ())),
                                   preferred_element_type=jnp.float32)

    for e in range(E):
        we = jnp.sum(cmb * (col == e).astype(jnp.float32),
                     axis=-1, keepdims=True)
        out = out + ffn(wg_ref[e], wu_ref[e], wd_ref[e]) * we
    out = out + ffn(sg_ref[...], su_ref[...], sd_ref[...])
    out_ref[...] = out


def _k4(hn2_bf, res2, cmb, w_gate, w_up, w_down, sh_gate, sh_up, sh_down):
    return pl.pallas_call(
        _k4_body,
        grid=(NBT,),
        in_specs=[
            pl.BlockSpec((BT, D), lambda i: (i, 0)),
            pl.BlockSpec((BT, D), lambda i: (i, 0)),
            pl.BlockSpec((BT, E), lambda i: (i, 0)),
            pl.BlockSpec((E, DFF, D), lambda i: (0, 0, 0)),
            pl.BlockSpec((E, DFF, D), lambda i: (0, 0, 0)),
            pl.BlockSpec((E, D, DFF), lambda i: (0, 0, 0)),
            pl.BlockSpec((DFF, D), lambda i: (0, 0)),
            pl.BlockSpec((DFF, D), lambda i: (0, 0)),
            pl.BlockSpec((D, DFF), lambda i: (0, 0)),
        ],
        out_specs=pl.BlockSpec((BT, D), lambda i: (i, 0)),
        out_shape=jax.ShapeDtypeStruct((T, D), jnp.float32),
    )(hn2_bf, res2, cmb, w_gate, w_up, w_down, sh_gate, sh_up, sh_down)


@jax.jit
def _run(hidden_states, qkv_w, attn_gate_w, o_w, q_norm_w, k_norm_w,
         ln1_w, ln2_w, router_w, expert_bias, w_gate, w_up, w_down,
         sh_gate, sh_up, sh_down):
    h = hidden_states
    qkv, gate = _k1(h, ln1_w, qkv_w, attn_gate_w)

    q = qkv[:, : NH * HD].reshape(T, NH, HD).transpose(1, 0, 2)
    k = qkv[:, NH * HD : NH * HD + NKV * HD].reshape(T, NKV, HD).transpose(1, 0, 2)
    v = qkv[:, NH * HD + NKV * HD :].reshape(T, NKV, HD).transpose(1, 0, 2)

    o = _k2(q, k, v, q_norm_w, k_norm_w)
    o2 = o.transpose(1, 0, 2).reshape(T, NH * HD)

    h2, hn2_bf, cmb = _k3(o2, gate, h, o_w, ln2_w, router_w, expert_bias)

    wg = w_gate.astype(jnp.bfloat16)
    wu = w_up.astype(jnp.bfloat16)
    wd = w_down.astype(jnp.bfloat16)
    sg = sh_gate.astype(jnp.bfloat16)
    su = sh_up.astype(jnp.bfloat16)
    sd = sh_down.astype(jnp.bfloat16)
    return _k4(hn2_bf, h2, cmb, wg, wu, wd, sg, su, sd)


def kernel(positions, hidden_states, qkv_w, attn_gate_w, o_w, q_norm_w,
           k_norm_w, ln1_w, ln2_w, router_w, expert_bias, w_gate, w_up,
           w_down, sh_gate, sh_up, sh_down):
    return _run(hidden_states, qkv_w, attn_gate_w, o_w, q_norm_w, k_norm_w,
                ln1_w, ln2_w, router_w, expert_bias, w_gate, w_up, w_down,
                sh_gate, sh_up, sh_down)


# speedup vs baseline: 1.7389x; 1.7389x over previous
import jax
import jax.numpy as jnp
from jax.experimental import pallas as pl

T = 2048
D = 1024
NH = 16
NKV = 4
HD = 64
E = 8
TOPK = 2
NG = 4
TG = 2
DFF = 512
EPS = 1e-05

BT = 256
NBT = T // BT
BQ = 512
NBQ = T // BQ

_HIGH = jax.lax.Precision.HIGHEST


def _rms(x, w, eps=EPS):
    v = jnp.mean(x * x, axis=-1, keepdims=True)
    return x * jax.lax.rsqrt(v + eps) * w


def _dot_nt(a, b):
    return jax.lax.dot_general(
        a.astype(jnp.bfloat16), b.astype(jnp.bfloat16),
        (((1,), (1,)), ((), ())), preferred_element_type=jnp.float32)


def _k1_body(h_ref, ln1_ref, qkvw_ref, gatew_ref, qn_ref, kn_ref,
             q_out, k_out, v_out, gate_out):
    x = h_ref[...]
    xn = _rms(x, ln1_ref[...])
    y = _dot_nt(xn, qkvw_ref[...])
    gate_out[...] = _dot_nt(xn, gatew_ref[...])
    qn = qn_ref[...]
    kn = kn_ref[...]
    for h in range(NH):
        qh = _rms(y[:, h * HD:(h + 1) * HD], qn) * (HD ** -0.5)
        q_out[h] = qh.astype(jnp.bfloat16)
    for h in range(NKV):
        kb = NH * HD + h * HD
        k_out[h] = _rms(y[:, kb:kb + HD], kn).astype(jnp.bfloat16)
        vb = (NH + NKV) * HD + h * HD
        v_out[h] = y[:, vb:vb + HD].astype(jnp.bfloat16)


def _k1(h, ln1_w, qkv_w, attn_gate_w, q_norm_w, k_norm_w):
    return pl.pallas_call(
        _k1_body,
        grid=(NBT,),
        in_specs=[
            pl.BlockSpec((BT, D), lambda i: (i, 0)),
            pl.BlockSpec((1, D), lambda i: (0, 0)),
            pl.BlockSpec(((NH + 2 * NKV) * HD, D), lambda i: (0, 0)),
            pl.BlockSpec((NH * HD, D), lambda i: (0, 0)),
            pl.BlockSpec((1, HD), lambda i: (0, 0)),
            pl.BlockSpec((1, HD), lambda i: (0, 0)),
        ],
        out_specs=[
            pl.BlockSpec((NH, BT, HD), lambda i: (0, i, 0)),
            pl.BlockSpec((NKV, BT, HD), lambda i: (0, i, 0)),
            pl.BlockSpec((NKV, BT, HD), lambda i: (0, i, 0)),
            pl.BlockSpec((BT, NH * HD), lambda i: (i, 0)),
        ],
        out_shape=[
            jax.ShapeDtypeStruct((NH, T, HD), jnp.bfloat16),
            jax.ShapeDtypeStruct((NKV, T, HD), jnp.bfloat16),
            jax.ShapeDtypeStruct((NKV, T, HD), jnp.bfloat16),
            jax.ShapeDtypeStruct((T, NH * HD), jnp.float32),
        ],
    )(h, ln1_w.reshape(1, D), qkv_w, attn_gate_w,
      q_norm_w.reshape(1, HD), k_norm_w.reshape(1, HD))


def _attn_body(q_ref, k_ref, v_ref, o_ref):
    i = pl.program_id(1)
    qb = q_ref[0]

    def step(s, carry):
        m, l, acc, vb = carry
        m_new = jnp.maximum(m, jnp.max(s, axis=-1, keepdims=True))
        alpha = jnp.exp(m - m_new)
        p = jnp.exp(s - m_new)
        l = l * alpha + jnp.sum(p, axis=-1, keepdims=True)
        acc = acc * alpha + jax.lax.dot_general(
            p.astype(jnp.bfloat16), vb, (((1,), (0,)), ((), ())),
            preferred_element_type=jnp.float32)
        return m_new, l, acc

    def body(j, carry):
        kb = k_ref[0, pl.ds(j * BQ, BQ), :]
        vb = v_ref[0, pl.ds(j * BQ, BQ), :]
        s = jax.lax.dot_general(qb, kb, (((1,), (1,)), ((), ())),
                                preferred_element_type=jnp.float32)
        return step(s, carry + (vb,))

    m0 = jnp.full((BQ, 1), -1e30, jnp.float32)
    l0 = jnp.zeros((BQ, 1), jnp.float32)
    a0 = jnp.zeros((BQ, HD), jnp.float32)
    m, l, acc = jax.lax.fori_loop(0, i, body, (m0, l0, a0))

    kb = k_ref[0, pl.ds(i * BQ, BQ), :]
    vb = v_ref[0, pl.ds(i * BQ, BQ), :]
    s = jax.lax.dot_general(qb, kb, (((1,), (1,)), ((), ())),
                            preferred_element_type=jnp.float32)
    rows = jax.lax.broadcasted_iota(jnp.int32, (BQ, BQ), 0)
    cols = jax.lax.broadcasted_iota(jnp.int32, (BQ, BQ), 1)
    s = jnp.where(cols <= rows, s, -1e30)
    m, l, acc = step(s, (m, l, acc, vb))

    o_ref[0] = acc / l


def _k2(q, k, v):
    rep = NH // NKV
    return pl.pallas_call(
        _attn_body,
        grid=(NH, NBQ),
        in_specs=[
            pl.BlockSpec((1, BQ, HD), lambda h, i: (h, i, 0)),
            pl.BlockSpec((1, T, HD), lambda h, i: (h // rep, 0, 0)),
            pl.BlockSpec((1, T, HD), lambda h, i: (h // rep, 0, 0)),
        ],
        out_specs=pl.BlockSpec((1, BQ, HD), lambda h, i: (h, i, 0)),
        out_shape=jax.ShapeDtypeStruct((NH, T, HD), jnp.float32),
    )(q, k, v)


def _rank_lt(m, kmax, n):
    rank = jnp.zeros_like(m)
    e_idx = jax.lax.broadcasted_iota(jnp.int32, m.shape, 1)
    for r in range(1, n):
        m_rot = jnp.concatenate([m[:, r:], m[:, :r]], axis=1)
        beat = (m_rot > m) | ((m_rot == m) & (e_idx >= n - r))
        rank = rank + beat.astype(jnp.float32)
    return (rank < kmax).astype(jnp.float32)


def _k3_body(o_ref, gate_ref, res_ref, ow_ref, ln2_ref, rw_ref, bias_ref,
             h2_out, hn2_out, cmb_out):
    og = o_ref[...] * jax.nn.sigmoid(gate_ref[...])
    h2 = _dot_nt(og, ow_ref[...]) + res_ref[...]
    h2_out[...] = h2
    hn2 = _rms(h2, ln2_ref[...])
    hn2_out[...] = hn2.astype(jnp.bfloat16)

    logits = jax.lax.dot_general(
        hn2, rw_ref[...], (((1,), (1,)), ((), ())),
        preferred_element_type=jnp.float32, precision=_HIGH)
    scores = jax.nn.sigmoid(logits)
    sfc = scores + bias_ref[...]

    epg = E // NG
    pa = jax.lax.broadcasted_iota(jnp.int32, (E, NG), 0)
    pg = jax.lax.broadcasted_iota(jnp.int32, (E, NG), 1)
    pair = ((pa // epg) == pg).astype(jnp.float32)
    gs = jax.lax.dot_general(
        sfc, pair, (((1,), (0,)), ((), ())),
        preferred_element_type=jnp.float32, precision=_HIGH)

    sel_g = _rank_lt(gs, TG, NG)
    ge = jax.lax.broadcasted_iota(jnp.int32, (NG, E), 0)
    ee = jax.lax.broadcasted_iota(jnp.int32, (NG, E), 1)
    expand = (ge == (ee // epg)).astype(jnp.float32)
    mask_e = jax.lax.dot_general(
        sel_g, expand, (((1,), (0,)), ((), ())),
        preferred_element_type=jnp.float32, precision=_HIGH)

    masked = jnp.where(mask_e > 0.5, sfc, -1e30)
    sel_e = _rank_lt(masked, TOPK, E)
    w = scores * sel_e
    denom = jnp.sum(w, axis=-1, keepdims=True) + 1e-20
    cmb_out[...] = w / denom


def _k3(o2, gate, res, o_w, ln2_w, router_w, expert_bias):
    return pl.pallas_call(
        _k3_body,
        grid=(NBT,),
        in_specs=[
            pl.BlockSpec((BT, D), lambda i: (i, 0)),
            pl.BlockSpec((BT, D), lambda i: (i, 0)),
            pl.BlockSpec((BT, D), lambda i: (i, 0)),
            pl.BlockSpec((D, D), lambda i: (0, 0)),
            pl.BlockSpec((1, D), lambda i: (0, 0)),
            pl.BlockSpec((E, D), lambda i: (0, 0)),
            pl.BlockSpec((1, E), lambda i: (0, 0)),
        ],
        out_specs=[
            pl.BlockSpec((BT, D), lambda i: (i, 0)),
            pl.BlockSpec((BT, D), lambda i: (i, 0)),
            pl.BlockSpec((BT, E), lambda i: (i, 0)),
        ],
        out_shape=[
            jax.ShapeDtypeStruct((T, D), jnp.float32),
            jax.ShapeDtypeStruct((T, D), jnp.bfloat16),
            jax.ShapeDtypeStruct((T, E), jnp.float32),
        ],
    )(o2, gate, res, o_w, ln2_w.reshape(1, D), router_w,
      expert_bias.reshape(1, E))


def _k4_body(x_ref, res_ref, cmb_ref, wg_ref, wu_ref, wd_ref,
             sg_ref, su_ref, sd_ref, out_ref):
    x = x_ref[...]
    cmb = cmb_ref[...]
    col = jax.lax.broadcasted_iota(jnp.int32, (BT, E), 1)
    out = res_ref[...]

    def ffn(wg, wu, wd):
        g = jax.lax.dot_general(x, wg, (((1,), (1,)), ((), ())),
                                preferred_element_type=jnp.float32)
        u = jax.lax.dot_general(x, wu, (((1,), (1,)), ((), ())),
                                preferred_element_type=jnp.float32)
        a = (g * jax.nn.sigmoid(g) * u).astype(jnp.bfloat16)
        return jax.lax.dot_general(a, wd, (((1,), (1,)), ((), ())),
                                   preferred_element_type=jnp.float32)

    for e in range(E):
        we = jnp.sum(cmb * (col == e).astype(jnp.float32),
                     axis=-1, keepdims=True)
        out = out + ffn(wg_ref[e], wu_ref[e], wd_ref[e]) * we
    out = out + ffn(sg_ref[...], su_ref[...], sd_ref[...])
    out_ref[...] = out


def _k4(hn2_bf, res2, cmb, w_gate, w_up, w_down, sh_gate, sh_up, sh_down):
    return pl.pallas_call(
        _k4_body,
        grid=(NBT,),
        in_specs=[
            pl.BlockSpec((BT, D), lambda i: (i, 0)),
            pl.BlockSpec((BT, D), lambda i: (i, 0)),
            pl.BlockSpec((BT, E), lambda i: (i, 0)),
            pl.BlockSpec((E, DFF, D), lambda i: (0, 0, 0)),
            pl.BlockSpec((E, DFF, D), lambda i: (0, 0, 0)),
            pl.BlockSpec((E, D, DFF), lambda i: (0, 0, 0)),
            pl.BlockSpec((DFF, D), lambda i: (0, 0)),
            pl.BlockSpec((DFF, D), lambda i: (0, 0)),
            pl.BlockSpec((D, DFF), lambda i: (0, 0)),
        ],
        out_specs=pl.BlockSpec((BT, D), lambda i: (i, 0)),
        out_shape=jax.ShapeDtypeStruct((T, D), jnp.float32),
    )(hn2_bf, res2, cmb, w_gate, w_up, w_down, sh_gate, sh_up, sh_down)


@jax.jit
def _run(hidden_states, qkv_w, attn_gate_w, o_w, q_norm_w, k_norm_w,
         ln1_w, ln2_w, router_w, expert_bias, w_gate, w_up, w_down,
         sh_gate, sh_up, sh_down):
    h = hidden_states
    q, k, v, gate = _k1(h, ln1_w, qkv_w, attn_gate_w, q_norm_w, k_norm_w)

    o = _k2(q, k, v)
    o2 = o.transpose(1, 0, 2).reshape(T, NH * HD)

    h2, hn2_bf, cmb = _k3(o2, gate, h, o_w, ln2_w, router_w, expert_bias)

    wg = w_gate.astype(jnp.bfloat16)
    wu = w_up.astype(jnp.bfloat16)
    wd = w_down.astype(jnp.bfloat16)
    sg = sh_gate.astype(jnp.bfloat16)
    su = sh_up.astype(jnp.bfloat16)
    sd = sh_down.astype(jnp.bfloat16)
    return _k4(hn2_bf, h2, cmb, wg, wu, wd, sg, su, sd)


def kernel(positions, hidden_states, qkv_w, attn_gate_w, o_w, q_norm_w,
           k_norm_w, ln1_w, ln2_w, router_w, expert_bias, w_gate, w_up,
           w_down, sh_gate, sh_up, sh_down):
    return _run(hidden_states, qkv_w, attn_gate_w, o_w, q_norm_w, k_norm_w,
                ln1_w, ln2_w, router_w, expert_bias, w_gate, w_up, w_down,
                sh_gate, sh_up, sh_down)


# speedup vs baseline: 1.8596x; 1.0694x over previous
import jax
import jax.numpy as jnp
from jax.experimental import pallas as pl

T = 2048
D = 1024
NH = 16
NKV = 4
HD = 64
E = 8
TOPK = 2
NG = 4
TG = 2
DFF = 512
EPS = 1e-05

BT = 256
NBT = T // BT
BQ = 512
NBQ = T // BQ

_HIGH = jax.lax.Precision.HIGHEST


def _rms(x, w, eps=EPS):
    v = jnp.mean(x * x, axis=-1, keepdims=True)
    return x * jax.lax.rsqrt(v + eps) * w


def _dot_nt(a, b):
    return jax.lax.dot_general(
        a.astype(jnp.bfloat16), b.astype(jnp.bfloat16),
        (((1,), (1,)), ((), ())), preferred_element_type=jnp.float32)


def _k1_body(h_ref, ln1_ref, qkvw_ref, gatew_ref, qn_ref, kn_ref,
             q_out, k_out, v_out, gate_out):
    x = h_ref[...]
    xn = _rms(x, ln1_ref[...])
    y = _dot_nt(xn, qkvw_ref[...])
    gate_out[...] = _dot_nt(xn, gatew_ref[...])
    qn = qn_ref[...]
    kn = kn_ref[...]
    for h in range(NH):
        qh = _rms(y[:, h * HD:(h + 1) * HD], qn) * (HD ** -0.5)
        q_out[h] = qh.astype(jnp.bfloat16)
    for h in range(NKV):
        kb = NH * HD + h * HD
        k_out[h] = _rms(y[:, kb:kb + HD], kn).astype(jnp.bfloat16)
        vb = (NH + NKV) * HD + h * HD
        v_out[h] = y[:, vb:vb + HD].astype(jnp.bfloat16)


def _k1(h, ln1_w, qkv_w, attn_gate_w, q_norm_w, k_norm_w):
    return pl.pallas_call(
        _k1_body,
        grid=(NBT,),
        in_specs=[
            pl.BlockSpec((BT, D), lambda i: (i, 0)),
            pl.BlockSpec((1, D), lambda i: (0, 0)),
            pl.BlockSpec(((NH + 2 * NKV) * HD, D), lambda i: (0, 0)),
            pl.BlockSpec((NH * HD, D), lambda i: (0, 0)),
            pl.BlockSpec((1, HD), lambda i: (0, 0)),
            pl.BlockSpec((1, HD), lambda i: (0, 0)),
        ],
        out_specs=[
            pl.BlockSpec((NH, BT, HD), lambda i: (0, i, 0)),
            pl.BlockSpec((NKV, BT, HD), lambda i: (0, i, 0)),
            pl.BlockSpec((NKV, BT, HD), lambda i: (0, i, 0)),
            pl.BlockSpec((BT, NH * HD), lambda i: (i, 0)),
        ],
        out_shape=[
            jax.ShapeDtypeStruct((NH, T, HD), jnp.bfloat16),
            jax.ShapeDtypeStruct((NKV, T, HD), jnp.bfloat16),
            jax.ShapeDtypeStruct((NKV, T, HD), jnp.bfloat16),
            jax.ShapeDtypeStruct((T, NH * HD), jnp.float32),
        ],
    )(h, ln1_w.reshape(1, D), qkv_w, attn_gate_w,
      q_norm_w.reshape(1, HD), k_norm_w.reshape(1, HD))


def _attn_body(q_ref, k_ref, v_ref, o_ref):
    i = pl.program_id(1)
    qb = q_ref[0]

    def pexp(s):
        return jnp.exp(jnp.minimum(s, 30.0))

    def body(j, carry):
        l, acc = carry
        kb = k_ref[0, pl.ds(j * BQ, BQ), :]
        vb = v_ref[0, pl.ds(j * BQ, BQ), :]
        s = jax.lax.dot_general(qb, kb, (((1,), (1,)), ((), ())),
                                preferred_element_type=jnp.float32)
        p = pexp(s)
        l = l + jnp.sum(p, axis=-1, keepdims=True)
        acc = acc + jax.lax.dot_general(
            p.astype(jnp.bfloat16), vb, (((1,), (0,)), ((), ())),
            preferred_element_type=jnp.float32)
        return l, acc

    l0 = jnp.zeros((BQ, 1), jnp.float32)
    a0 = jnp.zeros((BQ, HD), jnp.float32)
    l, acc = jax.lax.fori_loop(0, i, body, (l0, a0))

    rows = jax.lax.broadcasted_iota(jnp.int32, (BQ, BQ), 0)
    cols = jax.lax.broadcasted_iota(jnp.int32, (BQ, BQ), 1)
    maskf = (cols <= rows).astype(jnp.float32)
    kb = k_ref[0, pl.ds(i * BQ, BQ), :]
    vb = v_ref[0, pl.ds(i * BQ, BQ), :]
    s = jax.lax.dot_general(qb, kb, (((1,), (1,)), ((), ())),
                            preferred_element_type=jnp.float32)
    p = pexp(s) * maskf
    l = l + jnp.sum(p, axis=-1, keepdims=True)
    acc = acc + jax.lax.dot_general(
        p.astype(jnp.bfloat16), vb, (((1,), (0,)), ((), ())),
        preferred_element_type=jnp.float32)

    o_ref[0] = acc / l


def _k2(q, k, v):
    rep = NH // NKV
    return pl.pallas_call(
        _attn_body,
        grid=(NH, NBQ),
        in_specs=[
            pl.BlockSpec((1, BQ, HD), lambda h, i: (h, i, 0)),
            pl.BlockSpec((1, T, HD), lambda h, i: (h // rep, 0, 0)),
            pl.BlockSpec((1, T, HD), lambda h, i: (h // rep, 0, 0)),
        ],
        out_specs=pl.BlockSpec((1, BQ, HD), lambda h, i: (h, i, 0)),
        out_shape=jax.ShapeDtypeStruct((NH, T, HD), jnp.float32),
    )(q, k, v)


def _rank_lt(m, kmax, n):
    rank = jnp.zeros_like(m)
    e_idx = jax.lax.broadcasted_iota(jnp.int32, m.shape, 1)
    for r in range(1, n):
        m_rot = jnp.concatenate([m[:, r:], m[:, :r]], axis=1)
        beat = (m_rot > m) | ((m_rot == m) & (e_idx >= n - r))
        rank = rank + beat.astype(jnp.float32)
    return (rank < kmax).astype(jnp.float32)


def _k3_body(o_ref, gate_ref, res_ref, ow_ref, ln2_ref, rw_ref, bias_ref,
             h2_out, hn2_out, cmb_out):
    og = o_ref[...] * jax.nn.sigmoid(gate_ref[...])
    h2 = _dot_nt(og, ow_ref[...]) + res_ref[...]
    h2_out[...] = h2
    hn2 = _rms(h2, ln2_ref[...])
    hn2_out[...] = hn2.astype(jnp.bfloat16)

    logits = jax.lax.dot_general(
        hn2, rw_ref[...], (((1,), (1,)), ((), ())),
        preferred_element_type=jnp.float32, precision=_HIGH)
    scores = jax.nn.sigmoid(logits)
    sfc = scores + bias_ref[...]

    epg = E // NG
    pa = jax.lax.broadcasted_iota(jnp.int32, (E, NG), 0)
    pg = jax.lax.broadcasted_iota(jnp.int32, (E, NG), 1)
    pair = ((pa // epg) == pg).astype(jnp.float32)
    gs = jax.lax.dot_general(
        sfc, pair, (((1,), (0,)), ((), ())),
        preferred_element_type=jnp.float32, precision=_HIGH)

    sel_g = _rank_lt(gs, TG, NG)
    ge = jax.lax.broadcasted_iota(jnp.int32, (NG, E), 0)
    ee = jax.lax.broadcasted_iota(jnp.int32, (NG, E), 1)
    expand = (ge == (ee // epg)).astype(jnp.float32)
    mask_e = jax.lax.dot_general(
        sel_g, expand, (((1,), (0,)), ((), ())),
        preferred_element_type=jnp.float32, precision=_HIGH)

    masked = jnp.where(mask_e > 0.5, sfc, -1e30)
    sel_e = _rank_lt(masked, TOPK, E)
    w = scores * sel_e
    denom = jnp.sum(w, axis=-1, keepdims=True) + 1e-20
    cmb_out[...] = w / denom


def _k3(o2, gate, res, o_w, ln2_w, router_w, expert_bias):
    return pl.pallas_call(
        _k3_body,
        grid=(NBT,),
        in_specs=[
            pl.BlockSpec((BT, D), lambda i: (i, 0)),
            pl.BlockSpec((BT, D), lambda i: (i, 0)),
            pl.BlockSpec((BT, D), lambda i: (i, 0)),
            pl.BlockSpec((D, D), lambda i: (0, 0)),
            pl.BlockSpec((1, D), lambda i: (0, 0)),
            pl.BlockSpec((E, D), lambda i: (0, 0)),
            pl.BlockSpec((1, E), lambda i: (0, 0)),
        ],
        out_specs=[
            pl.BlockSpec((BT, D), lambda i: (i, 0)),
            pl.BlockSpec((BT, D), lambda i: (i, 0)),
            pl.BlockSpec((BT, E), lambda i: (i, 0)),
        ],
        out_shape=[
            jax.ShapeDtypeStruct((T, D), jnp.float32),
            jax.ShapeDtypeStruct((T, D), jnp.bfloat16),
            jax.ShapeDtypeStruct((T, E), jnp.float32),
        ],
    )(o2, gate, res, o_w, ln2_w.reshape(1, D), router_w,
      expert_bias.reshape(1, E))


def _k4_body(x_ref, res_ref, cmb_ref, wg_ref, wu_ref, wd_ref,
             sg_ref, su_ref, sd_ref, out_ref):
    x = x_ref[...]
    cmb = cmb_ref[...]
    col = jax.lax.broadcasted_iota(jnp.int32, (BT, E), 1)
    out = res_ref[...]

    def ffn(wg, wu, wd):
        g = jax.lax.dot_general(x, wg, (((1,), (1,)), ((), ())),
                                preferred_element_type=jnp.float32)
        u = jax.lax.dot_general(x, wu, (((1,), (1,)), ((), ())),
                                preferred_element_type=jnp.float32)
        a = (g * jax.nn.sigmoid(g) * u).astype(jnp.bfloat16)
        return jax.lax.dot_general(a, wd, (((1,), (1,)), ((), ())),
                                   preferred_element_type=jnp.float32)

    for e in range(E):
        we = jnp.sum(cmb * (col == e).astype(jnp.float32),
                     axis=-1, keepdims=True)
        out = out + ffn(wg_ref[e], wu_ref[e], wd_ref[e]) * we
    out = out + ffn(sg_ref[...], su_ref[...], sd_ref[...])
    out_ref[...] = out


def _k4(hn2_bf, res2, cmb, w_gate, w_up, w_down, sh_gate, sh_up, sh_down):
    return pl.pallas_call(
        _k4_body,
        grid=(NBT,),
        in_specs=[
            pl.BlockSpec((BT, D), lambda i: (i, 0)),
            pl.BlockSpec((BT, D), lambda i: (i, 0)),
            pl.BlockSpec((BT, E), lambda i: (i, 0)),
            pl.BlockSpec((E, DFF, D), lambda i: (0, 0, 0)),
            pl.BlockSpec((E, DFF, D), lambda i: (0, 0, 0)),
            pl.BlockSpec((E, D, DFF), lambda i: (0, 0, 0)),
            pl.BlockSpec((DFF, D), lambda i: (0, 0)),
            pl.BlockSpec((DFF, D), lambda i: (0, 0)),
            pl.BlockSpec((D, DFF), lambda i: (0, 0)),
        ],
        out_specs=pl.BlockSpec((BT, D), lambda i: (i, 0)),
        out_shape=jax.ShapeDtypeStruct((T, D), jnp.float32),
    )(hn2_bf, res2, cmb, w_gate, w_up, w_down, sh_gate, sh_up, sh_down)


@jax.jit
def _run(hidden_states, qkv_w, attn_gate_w, o_w, q_norm_w, k_norm_w,
         ln1_w, ln2_w, router_w, expert_bias, w_gate, w_up, w_down,
         sh_gate, sh_up, sh_down):
    h = hidden_states
    q, k, v, gate = _k1(h, ln1_w, qkv_w, attn_gate_w, q_norm_w, k_norm_w)

    o = _k2(q, k, v)
    o2 = o.transpose(1, 0, 2).reshape(T, NH * HD)

    h2, hn2_bf, cmb = _k3(o2, gate, h, o_w, ln2_w, router_w, expert_bias)

    wg = w_gate.astype(jnp.bfloat16)
    wu = w_up.astype(jnp.bfloat16)
    wd = w_down.astype(jnp.bfloat16)
    sg = sh_gate.astype(jnp.bfloat16)
    su = sh_up.astype(jnp.bfloat16)
    sd = sh_down.astype(jnp.bfloat16)
    return _k4(hn2_bf, h2, cmb, wg, wu, wd, sg, su, sd)


def kernel(positions, hidden_states, qkv_w, attn_gate_w, o_w, q_norm_w,
           k_norm_w, ln1_w, ln2_w, router_w, expert_bias, w_gate, w_up,
           w_down, sh_gate, sh_up, sh_down):
    return _run(hidden_states, qkv_w, attn_gate_w, o_w, q_norm_w, k_norm_w,
                ln1_w, ln2_w, router_w, expert_bias, w_gate, w_up, w_down,
                sh_gate, sh_up, sh_down)
